# baseline (speedup 1.0000x reference)
import functools

import jax
import jax.numpy as jnp
from jax.experimental import pallas as pl


def _split2(v):
    hi = v.astype(jnp.bfloat16)
    lo = (v - hi.astype(jnp.float32)).astype(jnp.bfloat16)
    return hi, lo


def _chamfer_block_kernel(x1_ref, x2tn_ref, d1_ref, d2_ref, *, mc):
    nb = pl.program_id(1)
    x1 = x1_ref[0]
    x2tn = x2tn_ref[0]
    n_blk = x1.shape[0]
    m = x2tn.shape[1]

    x1sq = x1[:, 0:1] ** 2 + x1[:, 1:2] ** 2 + x1[:, 2:3] ** 2
    x2sq = 0.25 * (x2tn[0:1, :] ** 2 + x2tn[1:2, :] ** 2 + x2tn[2:3, :] ** 2)
    h1, l1 = _split2(x1sq)
    h2, l2 = _split2(x2sq)

    bf = jnp.bfloat16
    ones_a = jnp.ones((n_blk, 2), bf)
    ones_b = jnp.ones((2, m), bf)
    a_aug = jnp.concatenate([x1.astype(bf), h1, l1, ones_a], axis=1)
    b_aug = jnp.concatenate(
        [x2tn.astype(bf), ones_b, h2, l2], axis=0)

    rowacc = None
    for mi in range(m // mc):
        sl = slice(mi * mc, (mi + 1) * mc)
        d = jnp.dot(a_aug, b_aug[:, sl], preferred_element_type=jnp.float32)
        rowacc = d if rowacc is None else jnp.minimum(rowacc, d)
        colchunk = jnp.min(d, axis=0)

        @pl.when(nb == 0)
        def _init():
            d2_ref[0, 0, sl] = colchunk

        @pl.when(nb != 0)
        def _acc():
            d2_ref[0, 0, sl] = jnp.minimum(d2_ref[0, 0, sl], colchunk)

    d1_ref[0, 0, :] = jnp.min(rowacc, axis=1)


@functools.partial(jax.jit, static_argnames=("nb", "mc"))
def _chamfer(input1, input2, nb=1024, mc=512):
    b, n, _ = input1.shape
    m = input2.shape[1]
    x2t = -2.0 * jnp.transpose(input2, (0, 2, 1))
    grid = (b, n // nb)
    return pl.pallas_call(
        functools.partial(_chamfer_block_kernel, mc=mc),
        grid=grid,
        in_specs=[
            pl.BlockSpec((1, nb, 3), lambda bi, ni: (bi, ni, 0)),
            pl.BlockSpec((1, 3, m), lambda bi, ni: (bi, 0, 0)),
        ],
        out_specs=[
            pl.BlockSpec((1, 1, nb), lambda bi, ni: (bi, 0, ni)),
            pl.BlockSpec((1, 1, m), lambda bi, ni: (bi, 0, 0)),
        ],
        out_shape=[
            jax.ShapeDtypeStruct((b, 1, n), jnp.float32),
            jax.ShapeDtypeStruct((b, 1, m), jnp.float32),
        ],
    )(input1, x2t)


def kernel(input1, input2):
    dist1, dist2 = _chamfer(input1, input2)
    return (dist1[:, 0, :], dist2[:, 0, :])


# speedup vs baseline: 1.1589x; 1.1589x over previous
import functools

import jax
import jax.numpy as jnp
from jax.experimental import pallas as pl


def _split2(v):
    hi = v.astype(jnp.bfloat16)
    lo = (v - hi.astype(jnp.float32)).astype(jnp.bfloat16)
    return hi, lo


def _chamfer_block_kernel(x1_ref, x2tn_ref, d1_ref, d2_ref, *, mc):
    nb = pl.program_id(1)
    x1 = x1_ref[0]
    x2tn = x2tn_ref[0]
    n_blk = x1.shape[0]
    m = x2tn.shape[1]

    x1sq = x1[:, 0:1] ** 2 + x1[:, 1:2] ** 2 + x1[:, 2:3] ** 2
    x2sq = 0.25 * (x2tn[0:1, :] ** 2 + x2tn[1:2, :] ** 2 + x2tn[2:3, :] ** 2)
    h1, l1 = _split2(x1sq)
    h2, l2 = _split2(x2sq)

    bf = jnp.bfloat16
    ones_a = jnp.ones((n_blk, 2), bf)
    ones_b = jnp.ones((2, m), bf)
    a_aug = jnp.concatenate([x1.astype(bf), h1, l1, ones_a], axis=1)
    b_aug = jnp.concatenate(
        [x2tn.astype(bf), ones_b, h2, l2], axis=0)

    d = jnp.dot(a_aug, b_aug, preferred_element_type=jnp.float32)
    d1_ref[0, 0, :] = jnp.min(d, axis=1)
    colmin = jnp.min(d, axis=0)

    @pl.when(nb == 0)
    def _init():
        d2_ref[0, 0, :] = colmin

    @pl.when(nb != 0)
    def _acc():
        d2_ref[0, 0, :] = jnp.minimum(d2_ref[0, 0, :], colmin)


@functools.partial(jax.jit, static_argnames=("nb", "mc"))
def _chamfer(input1, input2, nb=1024, mc=512):
    b, n, _ = input1.shape
    m = input2.shape[1]
    x2t = -2.0 * jnp.transpose(input2, (0, 2, 1))
    grid = (b, n // nb)
    return pl.pallas_call(
        functools.partial(_chamfer_block_kernel, mc=mc),
        grid=grid,
        in_specs=[
            pl.BlockSpec((1, nb, 3), lambda bi, ni: (bi, ni, 0)),
            pl.BlockSpec((1, 3, m), lambda bi, ni: (bi, 0, 0)),
        ],
        out_specs=[
            pl.BlockSpec((1, 1, nb), lambda bi, ni: (bi, 0, ni)),
            pl.BlockSpec((1, 1, m), lambda bi, ni: (bi, 0, 0)),
        ],
        out_shape=[
            jax.ShapeDtypeStruct((b, 1, n), jnp.float32),
            jax.ShapeDtypeStruct((b, 1, m), jnp.float32),
        ],
    )(input1, x2t)


def kernel(input1, input2):
    dist1, dist2 = _chamfer(input1, input2)
    return (dist1[:, 0, :], dist2[:, 0, :])


# speedup vs baseline: 1.3520x; 1.1667x over previous
import functools

import jax
import jax.numpy as jnp
from jax.experimental import pallas as pl
from jax.experimental.pallas import tpu as pltpu


def _chamfer_block_kernel(x1_ref, x2tn_ref, d1_ref, d2_ref, b_scr):
    ni = pl.program_id(1)
    bf = jnp.bfloat16
    x1 = x1_ref[0]
    n_blk = x1.shape[0]

    @pl.when(ni == 0)
    def _build_b():
        x2tn = x2tn_ref[0]
        x2sq = 0.25 * (
            x2tn[0:1, :] ** 2 + x2tn[1:2, :] ** 2 + x2tn[2:3, :] ** 2)
        h2 = x2sq.astype(bf)
        r = x2sq - h2.astype(jnp.float32)
        m2 = r.astype(bf)
        l2 = (r - m2.astype(jnp.float32)).astype(bf)
        b_scr[0:3, :] = x2tn.astype(bf)
        b_scr[3:5, :] = jnp.ones((2, x2tn.shape[1]), bf)
        b_scr[5:6, :] = h2
        b_scr[6:7, :] = m2
        b_scr[7:8, :] = l2

    x1sq = x1[:, 0:1] ** 2 + x1[:, 1:2] ** 2 + x1[:, 2:3] ** 2
    h1 = x1sq.astype(bf)
    l1 = (x1sq - h1.astype(jnp.float32)).astype(bf)
    a_aug = jnp.concatenate(
        [x1.astype(bf), h1, l1, jnp.ones((n_blk, 3), bf)], axis=1)

    d = jnp.dot(a_aug, b_scr[...], preferred_element_type=jnp.float32)
    d1_ref[0, :, 0] = jnp.min(d, axis=1)
    colmin = jnp.min(d, axis=0)

    @pl.when(ni == 0)
    def _init():
        d2_ref[0, 0, :] = colmin

    @pl.when(ni != 0)
    def _acc():
        d2_ref[0, 0, :] = jnp.minimum(d2_ref[0, 0, :], colmin)


@functools.partial(jax.jit, static_argnames=("nb",))
def _chamfer(input1, input2, nb=1024):
    b, n, _ = input1.shape
    m = input2.shape[1]
    x2t = -2.0 * jnp.transpose(input2, (0, 2, 1))
    grid = (b, n // nb)
    return pl.pallas_call(
        _chamfer_block_kernel,
        grid=grid,
        in_specs=[
            pl.BlockSpec((1, nb, 3), lambda bi, ni: (bi, ni, 0)),
            pl.BlockSpec((1, 3, m), lambda bi, ni: (bi, 0, 0)),
        ],
        out_specs=[
            pl.BlockSpec((1, nb, 1), lambda bi, ni: (bi, ni, 0)),
            pl.BlockSpec((1, 1, m), lambda bi, ni: (bi, 0, 0)),
        ],
        out_shape=[
            jax.ShapeDtypeStruct((b, n, 1), jnp.float32),
            jax.ShapeDtypeStruct((b, 1, m), jnp.float32),
        ],
        scratch_shapes=[pltpu.VMEM((8, m), jnp.bfloat16)],
    )(input1, x2t)


def kernel(input1, input2):
    dist1, dist2 = _chamfer(input1, input2)
    return (dist1[:, :, 0], dist2[:, 0, :])


# speedup vs baseline: 1.4083x; 1.0416x over previous
import functools

import jax
import jax.numpy as jnp
from jax.experimental import pallas as pl
from jax.experimental.pallas import tpu as pltpu


def _chamfer_block_kernel(x1_ref, x2tn_ref, d1_ref, d2_ref, b_scr):
    ni = pl.program_id(1)
    bf = jnp.bfloat16
    x1 = x1_ref[0]
    n_blk = x1.shape[0]

    @pl.when(ni == 0)
    def _build_b():
        x2tn = x2tn_ref[0]
        x2sq = 0.25 * (
            x2tn[0:1, :] ** 2 + x2tn[1:2, :] ** 2 + x2tn[2:3, :] ** 2)
        h2 = x2sq.astype(bf)
        r = x2sq - h2.astype(jnp.float32)
        m2 = r.astype(bf)
        l2 = (r - m2.astype(jnp.float32)).astype(bf)
        b_scr[0:3, :] = x2tn.astype(bf)
        b_scr[3:5, :] = jnp.ones((2, x2tn.shape[1]), bf)
        b_scr[5:6, :] = h2
        b_scr[6:7, :] = m2
        b_scr[7:8, :] = l2

    x1sq = x1[:, 0:1] ** 2 + x1[:, 1:2] ** 2 + x1[:, 2:3] ** 2
    h1 = x1sq.astype(bf)
    l1 = (x1sq - h1.astype(jnp.float32)).astype(bf)
    a_aug = jnp.concatenate(
        [x1.astype(bf), h1, l1, jnp.ones((n_blk, 3), bf)], axis=1)

    d = jnp.dot(a_aug, b_scr[...], preferred_element_type=jnp.float32)
    d1_ref[0, :, 0] = jnp.min(d, axis=1)
    colmin = jnp.min(d, axis=0)

    @pl.when(ni == 0)
    def _init():
        d2_ref[0, 0, :] = colmin

    @pl.when(ni != 0)
    def _acc():
        d2_ref[0, 0, :] = jnp.minimum(d2_ref[0, 0, :], colmin)


@functools.partial(jax.jit, static_argnames=("nb",))
def _chamfer(input1, input2, nb=2048):
    b, n, _ = input1.shape
    m = input2.shape[1]
    x2t = -2.0 * jnp.transpose(input2, (0, 2, 1))
    grid = (b, n // nb)
    return pl.pallas_call(
        _chamfer_block_kernel,
        grid=grid,
        in_specs=[
            pl.BlockSpec((1, nb, 3), lambda bi, ni: (bi, ni, 0)),
            pl.BlockSpec((1, 3, m), lambda bi, ni: (bi, 0, 0)),
        ],
        out_specs=[
            pl.BlockSpec((1, nb, 1), lambda bi, ni: (bi, ni, 0)),
            pl.BlockSpec((1, 1, m), lambda bi, ni: (bi, 0, 0)),
        ],
        out_shape=[
            jax.ShapeDtypeStruct((b, n, 1), jnp.float32),
            jax.ShapeDtypeStruct((b, 1, m), jnp.float32),
        ],
        scratch_shapes=[pltpu.VMEM((8, m), jnp.bfloat16)],
    )(input1, x2t)


def kernel(input1, input2):
    dist1, dist2 = _chamfer(input1, input2)
    return (dist1[:, :, 0], dist2[:, 0, :])


# speedup vs baseline: 1.4108x; 1.0018x over previous
import functools

import jax
import jax.numpy as jnp
from jax.experimental import pallas as pl
from jax.experimental.pallas import tpu as pltpu


def _chamfer_block_kernel(x1_ref, x2tn_ref, d1_ref, d2_ref, b_scr):
    ni = pl.program_id(1)
    bf = jnp.bfloat16
    x1 = x1_ref[0]
    n_blk = x1.shape[0]

    @pl.when(ni == 0)
    def _build_b():
        x2tn = x2tn_ref[0]
        x2sq = 0.25 * (
            x2tn[0:1, :] ** 2 + x2tn[1:2, :] ** 2 + x2tn[2:3, :] ** 2)
        h2 = x2sq.astype(bf)
        r = x2sq - h2.astype(jnp.float32)
        m2 = r.astype(bf)
        l2 = (r - m2.astype(jnp.float32)).astype(bf)
        b_scr[0:3, :] = x2tn.astype(bf)
        b_scr[3:5, :] = jnp.ones((2, x2tn.shape[1]), bf)
        b_scr[5:6, :] = h2
        b_scr[6:7, :] = m2
        b_scr[7:8, :] = l2

    x1sq = x1[:, 0:1] ** 2 + x1[:, 1:2] ** 2 + x1[:, 2:3] ** 2
    h1 = x1sq.astype(bf)
    l1 = (x1sq - h1.astype(jnp.float32)).astype(bf)
    a_aug = jnp.concatenate(
        [x1.astype(bf), h1, l1, jnp.ones((n_blk, 3), bf)], axis=1)

    m = x2tn_ref.shape[2]
    mc = 1024
    rowacc = None
    colmins = []
    for mi in range(m // mc):
        d = jnp.dot(a_aug, b_scr[:, mi * mc:(mi + 1) * mc],
                    preferred_element_type=jnp.float32)
        rowacc = d if rowacc is None else jnp.minimum(rowacc, d)
        colmins.append(jnp.min(d, axis=0))
    d1_ref[0, :, 0] = jnp.min(rowacc, axis=1)
    colmin = jnp.concatenate(colmins)

    @pl.when(ni == 0)
    def _init():
        d2_ref[0, 0, :] = colmin

    @pl.when(ni != 0)
    def _acc():
        d2_ref[0, 0, :] = jnp.minimum(d2_ref[0, 0, :], colmin)


@functools.partial(jax.jit, static_argnames=("nb",))
def _chamfer(input1, input2, nb=2048):
    b, n, _ = input1.shape
    m = input2.shape[1]
    x2t = -2.0 * jnp.transpose(input2, (0, 2, 1))
    grid = (b, n // nb)
    return pl.pallas_call(
        _chamfer_block_kernel,
        grid=grid,
        in_specs=[
            pl.BlockSpec((1, nb, 3), lambda bi, ni: (bi, ni, 0)),
            pl.BlockSpec((1, 3, m), lambda bi, ni: (bi, 0, 0)),
        ],
        out_specs=[
            pl.BlockSpec((1, nb, 1), lambda bi, ni: (bi, ni, 0)),
            pl.BlockSpec((1, 1, m), lambda bi, ni: (bi, 0, 0)),
        ],
        out_shape=[
            jax.ShapeDtypeStruct((b, n, 1), jnp.float32),
            jax.ShapeDtypeStruct((b, 1, m), jnp.float32),
        ],
        scratch_shapes=[pltpu.VMEM((8, m), jnp.bfloat16)],
    )(input1, x2t)


def kernel(input1, input2):
    dist1, dist2 = _chamfer(input1, input2)
    return (dist1[:, :, 0], dist2[:, 0, :])


# speedup vs baseline: 1.4481x; 1.0264x over previous
import functools

import jax
import jax.numpy as jnp
from jax.experimental import pallas as pl
from jax.experimental.pallas import tpu as pltpu


def _chamfer_batch_kernel(x1_ref, x2tn_ref, d1_ref, d2_ref, b_scr, *, mc):
    bf = jnp.bfloat16
    x1 = x1_ref[0]
    x2tn = x2tn_ref[0]
    n = x1.shape[0]
    m = x2tn.shape[1]

    x2sq = 0.25 * (x2tn[0:1, :] ** 2 + x2tn[1:2, :] ** 2 + x2tn[2:3, :] ** 2)
    h2 = x2sq.astype(bf)
    r = x2sq - h2.astype(jnp.float32)
    m2 = r.astype(bf)
    l2 = (r - m2.astype(jnp.float32)).astype(bf)
    b_scr[0:3, :] = x2tn.astype(bf)
    b_scr[3:5, :] = jnp.ones((2, m), bf)
    b_scr[5:6, :] = h2
    b_scr[6:7, :] = m2
    b_scr[7:8, :] = l2

    x1sq = x1[:, 0:1] ** 2 + x1[:, 1:2] ** 2 + x1[:, 2:3] ** 2
    h1 = x1sq.astype(bf)
    l1 = (x1sq - h1.astype(jnp.float32)).astype(bf)
    a_aug = jnp.concatenate(
        [x1.astype(bf), h1, l1, jnp.ones((n, 3), bf)], axis=1)

    rowacc = None
    colmins = []
    for mi in range(m // mc):
        d = jnp.dot(a_aug, b_scr[:, mi * mc:(mi + 1) * mc],
                    preferred_element_type=jnp.float32)
        rowacc = d if rowacc is None else jnp.minimum(rowacc, d)
        colmins.append(jnp.min(d, axis=0))
    d1_ref[0, :, 0] = jnp.min(rowacc, axis=1)
    d2_ref[0, 0, :] = jnp.concatenate(colmins)


@functools.partial(jax.jit, static_argnames=("mc",))
def _chamfer(input1, input2, mc=1024):
    b, n, _ = input1.shape
    m = input2.shape[1]
    x2t = -2.0 * jnp.transpose(input2, (0, 2, 1))
    return pl.pallas_call(
        functools.partial(_chamfer_batch_kernel, mc=mc),
        grid=(b,),
        in_specs=[
            pl.BlockSpec((1, n, 3), lambda bi: (bi, 0, 0)),
            pl.BlockSpec((1, 3, m), lambda bi: (bi, 0, 0)),
        ],
        out_specs=[
            pl.BlockSpec((1, n, 1), lambda bi: (bi, 0, 0)),
            pl.BlockSpec((1, 1, m), lambda bi: (bi, 0, 0)),
        ],
        out_shape=[
            jax.ShapeDtypeStruct((b, n, 1), jnp.float32),
            jax.ShapeDtypeStruct((b, 1, m), jnp.float32),
        ],
        scratch_shapes=[pltpu.VMEM((8, m), jnp.bfloat16)],
    )(input1, x2t)


def kernel(input1, input2):
    dist1, dist2 = _chamfer(input1, input2)
    return (dist1[:, :, 0], dist2[:, 0, :])
